# baseline (speedup 1.0000x reference)
import jax
import jax.numpy as jnp
from jax import lax
from jax.experimental import pallas as pl
from jax.experimental.pallas import tpu as pltpu
from jax.experimental.pallas import tpu_sc as plsc
import numpy as np

NB_KNN_K = (10, 20, 100, 200)
MAX_NEIGHBORS = 200
INV_T = 1.0 / 0.07
N_CLASSES = 1000

Q = 1024
D = 64
N_TRAIN = 100000
N_PAD = 100352
NT = 2048
CHUNK = 64
NCHUNK = N_PAD // CHUNK
WIN = 12544
NWIN = N_PAD // WIN
CAP = 384
NCV = CAP // 16
BUFN = 208
TOPV = BUFN // 16
NEG_INF = float("-inf")
BLK = 16


def _sim_body(a_ref, b_ref, sim_ref, cmax_ref):
    i = pl.program_id(0)
    s = lax.dot_general(
        a_ref[...], b_ref[...], (((1,), (1,)), ((), ())),
        preferred_element_type=jnp.float32,
    )
    col = lax.broadcasted_iota(jnp.int32, (Q, NT), 1) + i * NT
    s = jnp.where(col < N_TRAIN, s, NEG_INF)
    sim_ref[...] = s
    cmax_ref[...] = jnp.max(s.reshape(Q, NT // CHUNK, CHUNK), axis=2)[None]


def _tau_body(cmax_ref, tau_ref):
    x = cmax_ref[...]
    u = lax.bitcast_convert_type(x, jnp.uint32)
    u = jnp.where((u >> 31) == 1, ~u, u | jnp.uint32(0x80000000))
    acc = jnp.zeros((Q, 1), jnp.uint32)
    for b in range(31, -1, -1):
        trial = acc | jnp.uint32(1 << b)
        cnt = jnp.sum((u >= trial).astype(jnp.int32), axis=1, keepdims=True)
        acc = jnp.where(cnt >= MAX_NEIGHBORS, trial, acc)
    f = jnp.where(acc >= jnp.uint32(0x80000000), acc ^ jnp.uint32(0x80000000), ~acc)
    tau = lax.bitcast_convert_type(f, jnp.float32)
    tau_ref[...] = jnp.broadcast_to(tau, (Q, 16))


@jax.jit
def _similarity_and_tau(features_rank, train_features):
    tf = jnp.pad(train_features, ((0, N_PAD - N_TRAIN), (0, 0)))
    sim, cmax = pl.pallas_call(
        _sim_body,
        grid=(N_PAD // NT,),
        in_specs=[
            pl.BlockSpec((Q, D), lambda i: (0, 0)),
            pl.BlockSpec((NT, D), lambda i: (i, 0)),
        ],
        out_specs=[
            pl.BlockSpec((Q, NT), lambda i: (0, i)),
            pl.BlockSpec((1, Q, NT // CHUNK), lambda i: (i, 0, 0)),
        ],
        out_shape=[
            jax.ShapeDtypeStruct((Q, N_PAD), jnp.float32),
            jax.ShapeDtypeStruct((N_PAD // NT, Q, NT // CHUNK), jnp.float32),
        ],
    )(features_rank, tf)
    cmax2 = cmax.transpose(1, 0, 2).reshape(Q, NCHUNK)
    tau16 = pl.pallas_call(
        _tau_body,
        out_shape=jax.ShapeDtypeStruct((Q, 16), jnp.float32),
    )(cmax2)
    return sim, tau16


def _filter_body(sim_hbm, tau_hbm, cv_hbm, ci_hbm,
                 win0, win1, candv, candi, tauv, sem0, sem1):
    info = plsc.get_sparse_core_info()
    nc = info.num_cores
    wid = lax.axis_index("s") * nc + lax.axis_index("c")
    lane = lax.iota(jnp.int32, 16)
    zero16i = jnp.zeros((16,), jnp.int32)

    def row_body(r, _):
        q = r * 32 + wid
        pltpu.sync_copy(tau_hbm.at[q], tauv)
        for j in range(NCV + 1):
            candv[pl.ds(16 * j, 16)] = jnp.full((16,), NEG_INF, jnp.float32)
            candi[pl.ds(16 * j, 16)] = zero16i

        cp = pltpu.async_copy(sim_hbm.at[q, pl.ds(0, WIN)], win0, sem0)
        ptr = jnp.zeros((16,), jnp.int32)
        for w in range(NWIN):
            cur = win0 if w % 2 == 0 else win1
            cp.wait()
            if w + 1 < NWIN:
                nxt, nsem = (win1, sem1) if w % 2 == 0 else (win0, sem0)
                cp = pltpu.async_copy(
                    sim_hbm.at[q, pl.ds((w + 1) * WIN, WIN)], nxt, nsem)
            tv = tauv[...]

            def blk_body(ib, ptr, cur=cur, base=w * WIN, tv=tv):
                i0 = ib * BLK
                anym = cur[pl.ds(i0 * 16, 16)] >= tv
                for u in range(1, BLK):
                    anym = anym | (cur[pl.ds((i0 + u) * 16, 16)] >= tv)
                pca = plsc.all_reduce_population_count(anym)

                def append(p):
                    for u in range(BLK):
                        v = cur[pl.ds((i0 + u) * 16, 16)]
                        m = v >= tv
                        pc = plsc.all_reduce_population_count(m)
                        idx = lane + ((i0 + u) * 16 + base)
                        nsv, si = lax.sort((-v, idx), dimension=0,
                                           num_keys=1)
                        pos = jnp.minimum(p, CAP - 16) + lane
                        plsc.store_scatter(candv.at[:], [pos], -nsv)
                        plsc.store_scatter(candi.at[:], [pos], si)
                        p = p + pc
                    return p

                return lax.cond(pca[0] > 0, append, lambda p: p, ptr)

            ptr = lax.fori_loop(0, WIN // 16 // BLK, blk_body, ptr,
                                unroll=2)

        pltpu.sync_copy(candv.at[pl.ds(0, CAP)], cv_hbm.at[q])
        pltpu.sync_copy(candi.at[pl.ds(0, CAP)], ci_hbm.at[q])
        return 0

    lax.fori_loop(0, 32, row_body, 0)


@jax.jit
def _sc_filter(sim, tau16):
    mesh = plsc.VectorSubcoreMesh(core_axis_name="c", subcore_axis_name="s")
    f = pl.kernel(
        _filter_body,
        out_type=[
            jax.ShapeDtypeStruct((Q, CAP), jnp.float32),
            jax.ShapeDtypeStruct((Q, CAP), jnp.int32),
        ],
        mesh=mesh,
        compiler_params=pltpu.CompilerParams(needs_layout_passes=False),
        scratch_types=dict(
            win0=pltpu.VMEM((WIN,), jnp.float32),
            win1=pltpu.VMEM((WIN,), jnp.float32),
            candv=pltpu.VMEM((CAP + 16,), jnp.float32),
            candi=pltpu.VMEM((CAP + 16,), jnp.int32),
            tauv=pltpu.VMEM((16,), jnp.float32),
            sem0=pltpu.SemaphoreType.DMA,
            sem1=pltpu.SemaphoreType.DMA,
        ),
    )
    return f(sim, tau16)


RB = 128


def _rank_body(cv_ref, vc_ref, rank_ref):
    c = pl.program_id(0)
    full = cv_ref[...]
    vc = vc_ref[0]
    colg = lax.broadcasted_iota(jnp.int32, (RB, 16), 1) + c * 16
    a = full[:, None, :]
    b = vc[:, :, None]
    gt = (a > b).astype(jnp.int32)
    iota3 = lax.broadcasted_iota(jnp.int32, (RB, 16, CAP), 2)
    eqb = ((a == b) & (iota3 < colg[:, :, None])).astype(jnp.int32)
    rank_ref[...] = jnp.sum(gt + eqb, axis=2)[None]


def _weight_body(cv_ref, rank_ref, w_ref):
    v = cv_ref[...]
    r = rank_ref[...]
    mx = jnp.max(v, axis=1, keepdims=True)
    e = jnp.where(r < MAX_NEIGHBORS, jnp.exp((v - mx) * INV_T), 0.0)
    z = jnp.sum(e, axis=1, keepdims=True)
    w_ref[...] = e / z


@jax.jit
def _ranks_and_weights(cv):
    cv3 = cv.reshape(Q, NCV, 16).transpose(1, 0, 2)
    rank3 = pl.pallas_call(
        _rank_body,
        grid=(NCV, Q // RB),
        in_specs=[
            pl.BlockSpec((RB, CAP), lambda c, rb: (rb, 0)),
            pl.BlockSpec((1, RB, 16), lambda c, rb: (c, rb, 0)),
        ],
        out_specs=pl.BlockSpec((1, RB, 16), lambda c, rb: (c, rb, 0)),
        out_shape=jax.ShapeDtypeStruct((NCV, Q, 16), jnp.int32),
    )(cv, cv3)
    rank = rank3.transpose(1, 0, 2).reshape(Q, CAP)
    w = pl.pallas_call(
        _weight_body,
        out_shape=jax.ShapeDtypeStruct((Q, CAP), jnp.float32),
    )(cv, rank)
    return rank, w


_LANE = np.arange(16)


def _seg_lanes():
    bounds = [0] + list(NB_KNN_K)
    segs = []
    for s in range(4):
        lo, hi = bounds[s], bounds[s + 1]
        segs.append([(j, max(lo - 16 * j, 0), min(hi - 16 * j, 16))
                     for j in range(lo // 16, (hi + 15) // 16)])
    return segs


_SEGS = _seg_lanes()


def _vote_body(ci_hbm, rank_hbm, w_hbm, lab_hbm, out_hbm,
               cir, rr, wr, bufi, bufw, labv, acc, sem0, gsem):
    info = plsc.get_sparse_core_info()
    nc = info.num_cores
    wid = lax.axis_index("s") * nc + lax.axis_index("c")
    lane = lax.iota(jnp.int32, 16)
    zero16i = jnp.zeros((16,), jnp.int32)

    def row_body(r, _):
        q = r * 32 + wid
        pltpu.sync_copy(ci_hbm.at[q], cir)
        pltpu.sync_copy(rank_hbm.at[q], rr)
        pltpu.sync_copy(w_hbm.at[q], wr)
        for j in range(TOPV + 1):
            bufi[pl.ds(16 * j, 16)] = zero16i
            bufw[pl.ds(16 * j, 16)] = jnp.zeros((16,), jnp.float32)
        for j in range(NCV):
            r16 = rr[pl.ds(16 * j, 16)]
            m = r16 < BUFN
            plsc.store_scatter(bufw.at[:], [r16],
                               wr[pl.ds(16 * j, 16)], mask=m)
            plsc.store_scatter(bufi.at[:], [r16],
                               cir[pl.ds(16 * j, 16)], mask=m)
        cps = []
        for j in range(TOPV):
            idx = bufi[pl.ds(16 * j, 16)]
            cps.append(pltpu.async_copy(
                lab_hbm.at[idx], labv.at[pl.ds(16 * j, 16)], gsem))
        for c in cps:
            c.wait()
        for j in range(64):
            acc[pl.ds(16 * j, 16)] = jnp.zeros((16,), jnp.float32)
        for s in range(4):
            for j, klo, khi in _SEGS[s]:
                lab = labv[pl.ds(16 * j, 16)]
                wgt = bufw[pl.ds(16 * j, 16)]
                for k in range(klo, khi):
                    plsc.addupdate_scatter(acc.at[:], [lab], wgt,
                                           mask=lane == k)
            pltpu.sync_copy(acc.at[:], out_hbm.at[s, q])
        return 0

    lax.fori_loop(0, 32, row_body, 0)


@jax.jit
def _sc_vote(ci, rank, w, labels):
    mesh = plsc.VectorSubcoreMesh(core_axis_name="c", subcore_axis_name="s")
    f = pl.kernel(
        _vote_body,
        out_type=jax.ShapeDtypeStruct((4, Q, 1024), jnp.float32),
        mesh=mesh,
        compiler_params=pltpu.CompilerParams(needs_layout_passes=False),
        scratch_types=dict(
            cir=pltpu.VMEM((CAP,), jnp.int32),
            rr=pltpu.VMEM((CAP,), jnp.int32),
            wr=pltpu.VMEM((CAP,), jnp.float32),
            bufi=pltpu.VMEM((BUFN + 16,), jnp.int32),
            bufw=pltpu.VMEM((BUFN + 16,), jnp.float32),
            labv=pltpu.VMEM((16 * TOPV,), jnp.int32),
            acc=pltpu.VMEM((1024,), jnp.float32),
            sem0=pltpu.SemaphoreType.DMA,
            gsem=pltpu.SemaphoreType.DMA,
        ),
    )
    return f(ci, rank, w, labels)


def kernel(features_rank, train_features, train_labels):
    labels = train_labels.astype(jnp.int32)
    sim, tau16 = _similarity_and_tau(features_rank, train_features)
    cv, ci = _sc_filter(sim, tau16)
    rank, w = _ranks_and_weights(cv)
    probs = _sc_vote(ci, rank, w, labels)
    return tuple(probs[i, :, :N_CLASSES] for i in range(4))


# speedup vs baseline: 1.1464x; 1.1464x over previous
import jax
import jax.numpy as jnp
from jax import lax
from jax.experimental import pallas as pl
from jax.experimental.pallas import tpu as pltpu
from jax.experimental.pallas import tpu_sc as plsc
import numpy as np

NB_KNN_K = (10, 20, 100, 200)
MAX_NEIGHBORS = 200
INV_T = 1.0 / 0.07
N_CLASSES = 1000

Q = 1024
D = 64
N_TRAIN = 100000
N_PAD = 100352
NT = 2048
CHUNK = 64
NCHUNK = N_PAD // CHUNK
WIN = 12544
NWIN = N_PAD // WIN
CAP = 384
NCV = CAP // 16
BUFN = 208
TOPV = BUFN // 16
NEG_INF = float("-inf")
BLK = 8


def _sim_body(a_ref, b_ref, sim_ref, cmax_ref):
    i = pl.program_id(0)
    s = lax.dot_general(
        a_ref[...], b_ref[...], (((1,), (1,)), ((), ())),
        preferred_element_type=jnp.float32,
    )
    col = lax.broadcasted_iota(jnp.int32, (Q, NT), 1) + i * NT
    s = jnp.where(col < N_TRAIN, s, NEG_INF)
    sim_ref[...] = s
    cmax_ref[...] = jnp.max(s.reshape(Q, NT // CHUNK, CHUNK), axis=2)[None]


def _tau_body(cmax_ref, tau_ref):
    x = cmax_ref[...]
    u = lax.bitcast_convert_type(x, jnp.uint32)
    u = jnp.where((u >> 31) == 1, ~u, u | jnp.uint32(0x80000000))
    acc = jnp.zeros((Q, 1), jnp.uint32)
    for b in range(31, -1, -1):
        trial = acc | jnp.uint32(1 << b)
        cnt = jnp.sum((u >= trial).astype(jnp.int32), axis=1, keepdims=True)
        acc = jnp.where(cnt >= MAX_NEIGHBORS, trial, acc)
    f = jnp.where(acc >= jnp.uint32(0x80000000), acc ^ jnp.uint32(0x80000000), ~acc)
    tau = lax.bitcast_convert_type(f, jnp.float32)
    tau_ref[...] = jnp.broadcast_to(tau, (Q, 16))


@jax.jit
def _similarity_and_tau(features_rank, train_features):
    tf = jnp.pad(train_features, ((0, N_PAD - N_TRAIN), (0, 0)))
    sim, cmax = pl.pallas_call(
        _sim_body,
        grid=(N_PAD // NT,),
        in_specs=[
            pl.BlockSpec((Q, D), lambda i: (0, 0)),
            pl.BlockSpec((NT, D), lambda i: (i, 0)),
        ],
        out_specs=[
            pl.BlockSpec((Q, NT), lambda i: (0, i)),
            pl.BlockSpec((1, Q, NT // CHUNK), lambda i: (i, 0, 0)),
        ],
        out_shape=[
            jax.ShapeDtypeStruct((Q, N_PAD), jnp.float32),
            jax.ShapeDtypeStruct((N_PAD // NT, Q, NT // CHUNK), jnp.float32),
        ],
    )(features_rank, tf)
    cmax2 = cmax.transpose(1, 0, 2).reshape(Q, NCHUNK)
    tau16 = pl.pallas_call(
        _tau_body,
        out_shape=jax.ShapeDtypeStruct((Q, 16), jnp.float32),
    )(cmax2)
    return sim, tau16


def _filter_body(sim_hbm, tau_hbm, cv_hbm, ci_hbm,
                 win0, win1, candv, candi, tauv, sem0, sem1):
    info = plsc.get_sparse_core_info()
    nc = info.num_cores
    wid = lax.axis_index("s") * nc + lax.axis_index("c")
    lane = lax.iota(jnp.int32, 16)
    zero16i = jnp.zeros((16,), jnp.int32)

    def row_body(r, _):
        q = r * 32 + wid
        pltpu.sync_copy(tau_hbm.at[q], tauv)
        for j in range(NCV + 1):
            candv[pl.ds(16 * j, 16)] = jnp.full((16,), NEG_INF, jnp.float32)
            candi[pl.ds(16 * j, 16)] = zero16i

        cp = pltpu.async_copy(sim_hbm.at[q, pl.ds(0, WIN)], win0, sem0)
        ptr = jnp.zeros((16,), jnp.int32)
        for w in range(NWIN):
            cur = win0 if w % 2 == 0 else win1
            cp.wait()
            if w + 1 < NWIN:
                nxt, nsem = (win1, sem1) if w % 2 == 0 else (win0, sem0)
                cp = pltpu.async_copy(
                    sim_hbm.at[q, pl.ds((w + 1) * WIN, WIN)], nxt, nsem)
            tv = tauv[...]

            def blk_body(ib, ptr, cur=cur, base=w * WIN, tv=tv):
                i0 = ib * BLK
                anym = cur[pl.ds(i0 * 16, 16)] >= tv
                for u in range(1, BLK):
                    anym = anym | (cur[pl.ds((i0 + u) * 16, 16)] >= tv)
                pca = plsc.all_reduce_population_count(anym)

                def append(p):
                    for u in range(BLK):
                        v = cur[pl.ds((i0 + u) * 16, 16)]
                        m = v >= tv
                        pc = plsc.all_reduce_population_count(m)
                        idx = lane + ((i0 + u) * 16 + base)
                        nsv, si = lax.sort((-v, idx), dimension=0,
                                           num_keys=1)
                        pos = jnp.minimum(p, CAP - 16) + lane
                        plsc.store_scatter(candv.at[:], [pos], -nsv)
                        plsc.store_scatter(candi.at[:], [pos], si)
                        p = p + pc
                    return p

                return lax.cond(pca[0] > 0, append, lambda p: p, ptr)

            ptr = lax.fori_loop(0, WIN // 16 // BLK, blk_body, ptr,
                                unroll=2)

        pltpu.sync_copy(candv.at[pl.ds(0, CAP)], cv_hbm.at[q])
        pltpu.sync_copy(candi.at[pl.ds(0, CAP)], ci_hbm.at[q])
        return 0

    lax.fori_loop(0, 32, row_body, 0)


@jax.jit
def _sc_filter(sim, tau16):
    mesh = plsc.VectorSubcoreMesh(core_axis_name="c", subcore_axis_name="s")
    f = pl.kernel(
        _filter_body,
        out_type=[
            jax.ShapeDtypeStruct((Q, CAP), jnp.float32),
            jax.ShapeDtypeStruct((Q, CAP), jnp.int32),
        ],
        mesh=mesh,
        compiler_params=pltpu.CompilerParams(needs_layout_passes=False),
        scratch_types=dict(
            win0=pltpu.VMEM((WIN,), jnp.float32),
            win1=pltpu.VMEM((WIN,), jnp.float32),
            candv=pltpu.VMEM((CAP + 16,), jnp.float32),
            candi=pltpu.VMEM((CAP + 16,), jnp.int32),
            tauv=pltpu.VMEM((16,), jnp.float32),
            sem0=pltpu.SemaphoreType.DMA,
            sem1=pltpu.SemaphoreType.DMA,
        ),
    )
    return f(sim, tau16)


RB = 128


def _rank_body(cv_ref, vc_ref, rank_ref):
    c = pl.program_id(0)
    full = cv_ref[...]
    vc = vc_ref[0]
    colg = lax.broadcasted_iota(jnp.int32, (RB, 16), 1) + c * 16
    a = full[:, None, :]
    b = vc[:, :, None]
    gt = (a > b).astype(jnp.int32)
    iota3 = lax.broadcasted_iota(jnp.int32, (RB, 16, CAP), 2)
    eqb = ((a == b) & (iota3 < colg[:, :, None])).astype(jnp.int32)
    rank_ref[...] = jnp.sum(gt + eqb, axis=2)[None]


def _weight_body(cv_ref, rank_ref, w_ref):
    v = cv_ref[...]
    r = rank_ref[...]
    mx = jnp.max(v, axis=1, keepdims=True)
    e = jnp.where(r < MAX_NEIGHBORS, jnp.exp((v - mx) * INV_T), 0.0)
    z = jnp.sum(e, axis=1, keepdims=True)
    w_ref[...] = e / z


@jax.jit
def _ranks_and_weights(cv):
    cv3 = cv.reshape(Q, NCV, 16).transpose(1, 0, 2)
    rank3 = pl.pallas_call(
        _rank_body,
        grid=(NCV, Q // RB),
        in_specs=[
            pl.BlockSpec((RB, CAP), lambda c, rb: (rb, 0)),
            pl.BlockSpec((1, RB, 16), lambda c, rb: (c, rb, 0)),
        ],
        out_specs=pl.BlockSpec((1, RB, 16), lambda c, rb: (c, rb, 0)),
        out_shape=jax.ShapeDtypeStruct((NCV, Q, 16), jnp.int32),
    )(cv, cv3)
    rank = rank3.transpose(1, 0, 2).reshape(Q, CAP)
    w = pl.pallas_call(
        _weight_body,
        out_shape=jax.ShapeDtypeStruct((Q, CAP), jnp.float32),
    )(cv, rank)
    return rank, w


_LANE = np.arange(16)


def _seg_lanes():
    bounds = [0] + list(NB_KNN_K)
    segs = []
    for s in range(4):
        lo, hi = bounds[s], bounds[s + 1]
        segs.append([(j, max(lo - 16 * j, 0), min(hi - 16 * j, 16))
                     for j in range(lo // 16, (hi + 15) // 16)])
    return segs


_SEGS = _seg_lanes()


def _vote_body(ci_hbm, rank_hbm, w_hbm, lab_hbm, out_hbm,
               cir, rr, wr, bufi, bufw, labv, acc, sem0, gsem):
    info = plsc.get_sparse_core_info()
    nc = info.num_cores
    wid = lax.axis_index("s") * nc + lax.axis_index("c")
    lane = lax.iota(jnp.int32, 16)
    zero16i = jnp.zeros((16,), jnp.int32)

    def row_body(r, _):
        q = r * 32 + wid
        pltpu.sync_copy(ci_hbm.at[q], cir)
        pltpu.sync_copy(rank_hbm.at[q], rr)
        pltpu.sync_copy(w_hbm.at[q], wr)
        for j in range(TOPV + 1):
            bufi[pl.ds(16 * j, 16)] = zero16i
            bufw[pl.ds(16 * j, 16)] = jnp.zeros((16,), jnp.float32)
        for j in range(NCV):
            r16 = rr[pl.ds(16 * j, 16)]
            m = r16 < BUFN
            plsc.store_scatter(bufw.at[:], [r16],
                               wr[pl.ds(16 * j, 16)], mask=m)
            plsc.store_scatter(bufi.at[:], [r16],
                               cir[pl.ds(16 * j, 16)], mask=m)
        cps = []
        for j in range(TOPV):
            idx = bufi[pl.ds(16 * j, 16)]
            cps.append(pltpu.async_copy(
                lab_hbm.at[idx], labv.at[pl.ds(16 * j, 16)], gsem))
        for c in cps:
            c.wait()
        for j in range(64):
            acc[pl.ds(16 * j, 16)] = jnp.zeros((16,), jnp.float32)
        for s in range(4):
            for j, klo, khi in _SEGS[s]:
                lab = labv[pl.ds(16 * j, 16)]
                wgt = bufw[pl.ds(16 * j, 16)]
                for k in range(klo, khi):
                    plsc.addupdate_scatter(acc.at[:], [lab], wgt,
                                           mask=lane == k)
            pltpu.sync_copy(acc.at[:], out_hbm.at[s, q])
        return 0

    lax.fori_loop(0, 32, row_body, 0)


@jax.jit
def _sc_vote(ci, rank, w, labels):
    mesh = plsc.VectorSubcoreMesh(core_axis_name="c", subcore_axis_name="s")
    f = pl.kernel(
        _vote_body,
        out_type=jax.ShapeDtypeStruct((4, Q, 1024), jnp.float32),
        mesh=mesh,
        compiler_params=pltpu.CompilerParams(needs_layout_passes=False),
        scratch_types=dict(
            cir=pltpu.VMEM((CAP,), jnp.int32),
            rr=pltpu.VMEM((CAP,), jnp.int32),
            wr=pltpu.VMEM((CAP,), jnp.float32),
            bufi=pltpu.VMEM((BUFN + 16,), jnp.int32),
            bufw=pltpu.VMEM((BUFN + 16,), jnp.float32),
            labv=pltpu.VMEM((16 * TOPV,), jnp.int32),
            acc=pltpu.VMEM((1024,), jnp.float32),
            sem0=pltpu.SemaphoreType.DMA,
            gsem=pltpu.SemaphoreType.DMA,
        ),
    )
    return f(ci, rank, w, labels)


def kernel(features_rank, train_features, train_labels):
    labels = train_labels.astype(jnp.int32)
    sim, tau16 = _similarity_and_tau(features_rank, train_features)
    cv, ci = _sc_filter(sim, tau16)
    rank, w = _ranks_and_weights(cv)
    probs = _sc_vote(ci, rank, w, labels)
    return tuple(probs[i, :, :N_CLASSES] for i in range(4))
